# baseline (speedup 1.0000x reference)
import functools

import jax
import jax.numpy as jnp
from jax import lax
from jax.experimental import pallas as pl
from jax.experimental.pallas import tpu as pltpu
from jax.experimental.pallas import tpu_sc as plsc

B = 10000
E = 320000
DV = 128
DE = 16
H = 4
DT = 2 * DV + DE
FF = H * DV

NC = 2
NS = 16
CHUNK = 128
NCHUNKS = E // CHUNK
BASE_CH = NCHUNKS // (NC * NS)
REM_CH = NCHUNKS - BASE_CH * NC * NS
SC_BASE = NCHUNKS // NS
SC_REM = NCHUNKS - SC_BASE * NS
STRIPE = 624
STRIPE_TAIL = B - STRIPE * NS

KE = 2000
GE = E // KE
KN = 2000
GN = B // KN


def _lrelu(x):
    return jnp.where(x >= 0, x, 0.2 * x)


def _sc_gather_body(node_hbm, idx1_hbm, idx2_hbm, g1_hbm, g2_hbm,
                    idx_v0, idx_v1, row_v0, row_v1,
                    isem0, isem1, gsem0, gsem1, wsem0, wsem1):
    cid = lax.axis_index("c")
    sid = lax.axis_index("s")
    wid = sid * NC + cid
    NW = NC * NS
    idx_v = (idx_v0, idx_v1)
    row_v = (row_v0, row_v1)
    isem = (isem0, isem1)
    gsem = (gsem0, gsem1)
    wsem = (wsem0, wsem1)
    npair = BASE_CH // 2

    for idx_hbm, out_hbm in ((idx1_hbm, g1_hbm), (idx2_hbm, g2_hbm)):
        def load_idx(p, j):
            base = (j * NW + wid) * CHUNK
            pltpu.async_copy(idx_hbm.at[pl.ds(base, CHUNK)], idx_v[p], isem[p])

        load_idx(0, 0)
        load_idx(1, 1)

        def body(jj, _):
            for p in (0, 1):
                pltpu.make_async_copy(idx_hbm.at[pl.ds(0, CHUNK)],
                                      idx_v[p], isem[p]).wait()

                @pl.when(jj > 0)
                def _():
                    base = ((2 * jj + p - 2) * NW + wid) * CHUNK
                    pltpu.make_async_copy(row_v[p], out_hbm.at[pl.ds(base, CHUNK)],
                                          wsem[p]).wait()
                pltpu.async_copy(node_hbm.at[idx_v[p]], row_v[p], gsem[p])
            for p in (0, 1):
                pltpu.make_async_copy(node_hbm.at[idx_v[p]], row_v[p],
                                      gsem[p]).wait()
                base = ((2 * jj + p) * NW + wid) * CHUNK
                pltpu.async_copy(row_v[p], out_hbm.at[pl.ds(base, CHUNK)], wsem[p])
                load_idx(p, jnp.minimum(2 * jj + p + 2, BASE_CH - 1))
            return 0

        lax.fori_loop(0, npair, body, 0)
        for p in (0, 1):
            base = ((BASE_CH - 2 + p) * NW + wid) * CHUNK
            pltpu.make_async_copy(row_v[p], out_hbm.at[pl.ds(base, CHUNK)],
                                  wsem[p]).wait()
            pltpu.make_async_copy(idx_hbm.at[pl.ds(0, CHUNK)],
                                  idx_v[p], isem[p]).wait()

        @pl.when(wid < REM_CH)
        def _():
            base = (BASE_CH * NW + wid) * CHUNK
            pltpu.sync_copy(idx_hbm.at[pl.ds(base, CHUNK)], idx_v0)
            pltpu.async_copy(node_hbm.at[idx_v0], row_v0, gsem0).wait()
            pltpu.sync_copy(row_v0, out_hbm.at[pl.ds(base, CHUNK)])


@functools.partial(jax.jit, static_argnames=())
def _sc_gather(node_fea, idx1, idx2):
    mesh = plsc.VectorSubcoreMesh(core_axis_name="c", subcore_axis_name="s",
                                  num_cores=NC, num_subcores=NS)
    return pl.kernel(
        _sc_gather_body,
        out_type=(jax.ShapeDtypeStruct((E, DV), jnp.float32),
                  jax.ShapeDtypeStruct((E, DV), jnp.float32)),
        mesh=mesh,
        scratch_types=[
            pltpu.VMEM((CHUNK,), jnp.int32),
            pltpu.VMEM((CHUNK,), jnp.int32),
            pltpu.VMEM((CHUNK, DV), jnp.float32),
            pltpu.VMEM((CHUNK, DV), jnp.float32),
            pltpu.SemaphoreType.DMA,
            pltpu.SemaphoreType.DMA,
            pltpu.SemaphoreType.DMA,
            pltpu.SemaphoreType.DMA,
            pltpu.SemaphoreType.DMA,
            pltpu.SemaphoreType.DMA,
        ],
    )(node_fea, idx1, idx2)


def _tc1_body(g1, g2, ef, w1a, w1b, w1c, b1, w2, b2, w3, b3,
              h_out, st_out):
    i = pl.program_id(0)
    x = jnp.dot(g1[...], w1a[...], preferred_element_type=jnp.float32)
    x += jnp.dot(g2[...], w1b[...], preferred_element_type=jnp.float32)
    x += jnp.dot(ef[...], w1c[...], preferred_element_type=jnp.float32)
    x = _lrelu(x + b1[...])
    x = _lrelu(jnp.dot(x, w2[...], preferred_element_type=jnp.float32) + b2[...])
    h = jnp.dot(x, w3[...], preferred_element_type=jnp.float32) + b3[...]
    h_out[...] = h

    @pl.when(i == 0)
    def _():
        st_out[...] = jnp.zeros_like(st_out)
    st_out[0:1, :] += jnp.sum(h, axis=0, keepdims=True)
    st_out[1:2, :] += jnp.sum(h * h, axis=0, keepdims=True)


def _tc1(g1, g2, edge_fea, pw):
    wspec = lambda shape: pl.BlockSpec(shape, lambda i: (0, 0))
    return pl.pallas_call(
        _tc1_body,
        grid=(GE,),
        in_specs=[
            pl.BlockSpec((KE, DV), lambda i: (i, 0)),
            pl.BlockSpec((KE, DV), lambda i: (i, 0)),
            pl.BlockSpec((KE, DE), lambda i: (i, 0)),
            wspec((DV, DE)), wspec((DV, DE)), wspec((DE, DE)), wspec((1, DE)),
            wspec((DE, DE)), wspec((1, DE)),
            wspec((DE, DE)), wspec((1, DE)),
        ],
        out_specs=[
            pl.BlockSpec((KE, DE), lambda i: (i, 0)),
            pl.BlockSpec((2, DE), lambda i: (0, 0)),
        ],
        out_shape=[
            jax.ShapeDtypeStruct((E, DE), jnp.float32),
            jax.ShapeDtypeStruct((2, DE), jnp.float32),
        ],
        compiler_params=pltpu.CompilerParams(
            dimension_semantics=("arbitrary",)),
    )(g1, g2, edge_fea, *pw)


def _tc2_body(g1, g2, h, ef, st, bnw, bnb,
              a1a, a1b, a1e, ab1, a2, ab2,
              m1a, m1b, m1e, mb1, m2, mb2,
              x_out, enew_out):
    mean = st[0:1, :] * (1.0 / E)
    var = st[1:2, :] * (1.0 / E) - mean * mean
    inv = lax.rsqrt(var + 1e-5)
    scale = bnw[...] * inv
    shift = bnb[...] - mean * scale

    ek = h[...] * scale + shift
    enew_out[...] = ef[...] + ek

    bf = jnp.bfloat16
    g1b, g2b, ekb = g1[...].astype(bf), g2[...].astype(bf), ek.astype(bf)
    pre = jnp.dot(g1b, a1a[...], preferred_element_type=jnp.float32)
    pre += jnp.dot(g2b, a1b[...], preferred_element_type=jnp.float32)
    pre += jnp.dot(ekb, a1e[...], preferred_element_type=jnp.float32)
    a = _lrelu(pre + ab1[...]).astype(bf)
    sij = jnp.dot(a, a2[...], preferred_element_type=jnp.float32) + ab2[...]
    ex = jnp.exp(sij)

    pre = jnp.dot(g1b, m1a[...], preferred_element_type=jnp.float32)
    pre += jnp.dot(g2b, m1b[...], preferred_element_type=jnp.float32)
    pre += jnp.dot(ekb, m1e[...], preferred_element_type=jnp.float32)
    m = _lrelu(pre + mb1[...]).astype(bf)
    mij = jnp.dot(m, m2[...], preferred_element_type=jnp.float32) + mb2[...]
    exm = ex * mij

    for g in range(H):
        x_out[g, :, :] = ex[:, g * DV:(g + 1) * DV]
        x_out[H + g, :, :] = exm[:, g * DV:(g + 1) * DV]


def _tc2(g1, g2, h, edge_fea, stats, pw):
    wspec = lambda shape: pl.BlockSpec(shape, lambda i: (0, 0))
    return pl.pallas_call(
        _tc2_body,
        grid=(GE,),
        in_specs=[
            pl.BlockSpec((KE, DV), lambda i: (i, 0)),
            pl.BlockSpec((KE, DV), lambda i: (i, 0)),
            pl.BlockSpec((KE, DE), lambda i: (i, 0)),
            pl.BlockSpec((KE, DE), lambda i: (i, 0)),
            wspec((2, DE)), wspec((1, DE)), wspec((1, DE)),
            wspec((DV, DV)), wspec((DV, DV)), wspec((DE, DV)), wspec((1, DV)),
            wspec((DV, FF)), wspec((1, FF)),
            wspec((DV, DV)), wspec((DV, DV)), wspec((DE, DV)), wspec((1, DV)),
            wspec((DV, FF)), wspec((1, FF)),
        ],
        out_specs=[
            pl.BlockSpec((2 * H, KE, DV), lambda i: (0, i, 0)),
            pl.BlockSpec((KE, DE), lambda i: (i, 0)),
        ],
        out_shape=[
            jax.ShapeDtypeStruct((2 * H, E, DV), jnp.float32),
            jax.ShapeDtypeStruct((E, DE), jnp.float32),
        ],
        compiler_params=pltpu.CompilerParams(
            dimension_semantics=("arbitrary",)),
    )(g1, g2, h, edge_fea, stats, *pw)


def _sc_scatter_body(x_hbm, idx_hbm, zeros_hbm, dn_hbm,
                     idx_v0, idx_v1, data_v0, data_v1, acc,
                     isem0, isem1, dsem0, dsem1, ssem0, ssem1):
    cid = lax.axis_index("c")
    sid = lax.axis_index("s")
    idx_v = (idx_v0, idx_v1)
    data_v = (data_v0, data_v1)
    isem = (isem0, isem1)
    dsem = (dsem0, dsem1)
    ssem = (ssem0, ssem1)
    npair = SC_BASE // 2

    for k in range(H):
        g = cid * H + k
        pltpu.sync_copy(zeros_hbm, acc.at[pl.ds(sid * STRIPE, STRIPE)])

        @pl.when(sid == NS - 1)
        def _():
            pltpu.sync_copy(zeros_hbm.at[pl.ds(0, STRIPE_TAIL)],
                            acc.at[pl.ds(STRIPE * NS, STRIPE_TAIL)])
        plsc.subcore_barrier()

        def load(p, j):
            base = (j * NS + sid) * CHUNK
            pltpu.async_copy(idx_hbm.at[pl.ds(base, CHUNK)], idx_v[p], isem[p])
            pltpu.async_copy(x_hbm.at[g, pl.ds(base, CHUNK)], data_v[p], dsem[p])

        def wait_load(p):
            pltpu.make_async_copy(idx_hbm.at[pl.ds(0, CHUNK)],
                                  idx_v[p], isem[p]).wait()
            pltpu.make_async_copy(x_hbm.at[g, pl.ds(0, CHUNK)],
                                  data_v[p], dsem[p]).wait()

        load(0, 0)
        load(1, 1)

        def body(jj, _):
            wait_load(0)
            pltpu.async_copy(data_v0, acc.at[idx_v0], ssem0, add=True)
            wait_load(1)
            pltpu.async_copy(data_v1, acc.at[idx_v1], ssem1, add=True)
            pltpu.make_async_copy(data_v0, acc.at[idx_v0], ssem0).wait()
            load(0, jnp.minimum(2 * jj + 2, SC_BASE - 1))
            pltpu.make_async_copy(data_v1, acc.at[idx_v1], ssem1).wait()
            load(1, jnp.minimum(2 * jj + 3, SC_BASE - 1))
            return 0

        lax.fori_loop(0, npair, body, 0)
        wait_load(0)
        wait_load(1)

        @pl.when(sid < SC_REM)
        def _():
            base = (SC_BASE * NS + sid) * CHUNK
            pltpu.sync_copy(idx_hbm.at[pl.ds(base, CHUNK)], idx_v0)
            pltpu.sync_copy(x_hbm.at[g, pl.ds(base, CHUNK)], data_v0)
            pltpu.sync_copy(data_v0, acc.at[idx_v0], add=True)
        plsc.subcore_barrier()

        pltpu.sync_copy(acc.at[pl.ds(sid * STRIPE, STRIPE)],
                        dn_hbm.at[g, pl.ds(sid * STRIPE, STRIPE)])

        @pl.when(sid == NS - 1)
        def _():
            pltpu.sync_copy(acc.at[pl.ds(STRIPE * NS, STRIPE_TAIL)],
                            dn_hbm.at[g, pl.ds(STRIPE * NS, STRIPE_TAIL)])
        plsc.subcore_barrier()


def _sc_scatter(x, idx1, zeros):
    mesh = plsc.VectorSubcoreMesh(core_axis_name="c", subcore_axis_name="s",
                                  num_cores=NC, num_subcores=NS)
    return pl.kernel(
        _sc_scatter_body,
        out_type=jax.ShapeDtypeStruct((2 * H, B, DV), jnp.float32),
        mesh=mesh,
        scratch_types=[
            pltpu.VMEM((CHUNK,), jnp.int32),
            pltpu.VMEM((CHUNK,), jnp.int32),
            pltpu.VMEM((CHUNK, DV), jnp.float32),
            pltpu.VMEM((CHUNK, DV), jnp.float32),
            pltpu.VMEM_SHARED((B, DV), jnp.float32),
            pltpu.SemaphoreType.DMA,
            pltpu.SemaphoreType.DMA,
            pltpu.SemaphoreType.DMA,
            pltpu.SemaphoreType.DMA,
            pltpu.SemaphoreType.DMA,
            pltpu.SemaphoreType.DMA,
        ],
    )(x, idx1, zeros)


def _tc3a_body(dn, wo, wob, y_out, st_out):
    i = pl.program_id(0)
    cols = []
    for g in range(H):
        den = dn[g, :, :]
        num = dn[H + g, :, :]
        cols.append(jnp.where(den > 0, num / den, 0.0))
    msg = jnp.concatenate(cols, axis=1)
    y = jnp.dot(msg, wo[...], preferred_element_type=jnp.float32) + wob[...]
    y_out[...] = y

    @pl.when(i == 0)
    def _():
        st_out[...] = jnp.zeros_like(st_out)
    st_out[0:1, :] += jnp.sum(y, axis=0, keepdims=True)
    st_out[1:2, :] += jnp.sum(y * y, axis=0, keepdims=True)


def _tc3a(dn, woT, wob):
    wspec = lambda shape: pl.BlockSpec(shape, lambda i: (0, 0))
    return pl.pallas_call(
        _tc3a_body,
        grid=(GN,),
        in_specs=[
            pl.BlockSpec((2 * H, KN, DV), lambda i: (0, i, 0)),
            wspec((FF, DV)), wspec((1, DV)),
        ],
        out_specs=[
            pl.BlockSpec((KN, DV), lambda i: (i, 0)),
            pl.BlockSpec((2, DV), lambda i: (0, 0)),
        ],
        out_shape=[
            jax.ShapeDtypeStruct((B, DV), jnp.float32),
            jax.ShapeDtypeStruct((2, DV), jnp.float32),
        ],
        compiler_params=pltpu.CompilerParams(
            dimension_semantics=("arbitrary",)),
    )(dn, woT, wob)


def _tc3b_body(y, node, st, bnw, bnb, out):
    mean = st[0:1, :] * (1.0 / B)
    var = st[1:2, :] * (1.0 / B) - mean * mean
    inv = lax.rsqrt(var + 1e-5)
    scale = bnw[...] * inv
    shift = bnb[...] - mean * scale
    out[...] = node[...] + y[...] * scale + shift


def _tc3b(y, node_fea, stats, bnw, bnb):
    wspec = lambda shape: pl.BlockSpec(shape, lambda i: (0, 0))
    return pl.pallas_call(
        _tc3b_body,
        grid=(GN,),
        in_specs=[
            pl.BlockSpec((KN, DV), lambda i: (i, 0)),
            pl.BlockSpec((KN, DV), lambda i: (i, 0)),
            wspec((2, DV)), wspec((1, DV)), wspec((1, DV)),
        ],
        out_specs=pl.BlockSpec((KN, DV), lambda i: (i, 0)),
        out_shape=jax.ShapeDtypeStruct((B, DV), jnp.float32),
        compiler_params=pltpu.CompilerParams(
            dimension_semantics=("arbitrary",)),
    )(y, node_fea, stats, bnw, bnb)


def kernel(node_fea, idx1, idx2, edge_fea,
           pe_w1, pe_b1, pe_w2, pe_b2, pe_w3, pe_b3,
           fa_w1, fa_b1, fa_w2, fa_b2,
           fm_w1, fm_b1, fm_w2, fm_b2,
           wo_w, wo_b, bn1_w, bn1_b, bn2_w, bn2_b):
    f32 = jnp.float32
    idx1 = idx1.astype(jnp.int32)
    idx2 = idx2.astype(jnp.int32)

    g1, g2 = _sc_gather(node_fea, idx1, idx2)

    pe = (pe_w1[:, :DV].T, pe_w1[:, DV:2 * DV].T, pe_w1[:, 2 * DV:].T,
          pe_b1.reshape(1, DE),
          pe_w2.T, pe_b2.reshape(1, DE),
          pe_w3.T, pe_b3.reshape(1, DE))
    h, stats1 = _tc1(g1, g2, edge_fea, pe)

    bf = jnp.bfloat16
    mw = (fa_w1[:, :DV].T.astype(bf), fa_w1[:, DV:2 * DV].T.astype(bf),
          fa_w1[:, 2 * DV:].T.astype(bf),
          fa_b1.reshape(1, DV), fa_w2.T.astype(bf), fa_b2.reshape(1, FF),
          fm_w1[:, :DV].T.astype(bf), fm_w1[:, DV:2 * DV].T.astype(bf),
          fm_w1[:, 2 * DV:].T.astype(bf),
          fm_b1.reshape(1, DV), fm_w2.T.astype(bf), fm_b2.reshape(1, FF))
    x, edge_new = _tc2(g1, g2, h, edge_fea, stats1,
                       (bn1_w.reshape(1, DE), bn1_b.reshape(1, DE)) + mw)

    zeros = jnp.zeros((STRIPE, DV), f32)
    dn = _sc_scatter(x, idx1, zeros)

    y, stats2 = _tc3a(dn, wo_w.T, wo_b.reshape(1, DV))
    node_new = _tc3b(y, node_fea, stats2,
                     bn2_w.reshape(1, DV), bn2_b.reshape(1, DV))
    return (node_new, edge_new)


# speedup vs baseline: 1.0491x; 1.0491x over previous
import functools

import jax
import jax.numpy as jnp
from jax import lax
from jax.experimental import pallas as pl
from jax.experimental.pallas import tpu as pltpu
from jax.experimental.pallas import tpu_sc as plsc

B = 10000
E = 320000
DV = 128
DE = 16
H = 4
DT = 2 * DV + DE
FF = H * DV

NC = 2
NS = 16
CHUNK = 128
NCHUNKS = E // CHUNK
BASE_CH = NCHUNKS // (NC * NS)
REM_CH = NCHUNKS - BASE_CH * NC * NS
EH = E // 2
NCHUNKS_H = EH // CHUNK
SC_BASE = NCHUNKS_H // NS
SC_REM = NCHUNKS_H - SC_BASE * NS
STRIPE = 624
STRIPE_TAIL = B - STRIPE * NS

KE = 2000
GE = E // KE
KN = 1000
GN = B // KN


def _lrelu(x):
    return jnp.where(x >= 0, x, 0.2 * x)


def _sc_gather_body(node_hbm, idx1_hbm, idx2_hbm, g1_hbm, g2_hbm,
                    idx_v0, idx_v1, row_v0, row_v1,
                    isem0, isem1, gsem0, gsem1, wsem0, wsem1):
    cid = lax.axis_index("c")
    sid = lax.axis_index("s")
    wid = sid * NC + cid
    NW = NC * NS
    idx_v = (idx_v0, idx_v1)
    row_v = (row_v0, row_v1)
    isem = (isem0, isem1)
    gsem = (gsem0, gsem1)
    wsem = (wsem0, wsem1)
    npair = BASE_CH // 2

    for idx_hbm, out_hbm in ((idx1_hbm, g1_hbm), (idx2_hbm, g2_hbm)):
        def load_idx(p, j):
            base = (j * NW + wid) * CHUNK
            pltpu.async_copy(idx_hbm.at[pl.ds(base, CHUNK)], idx_v[p], isem[p])

        load_idx(0, 0)
        load_idx(1, 1)

        def body(jj, _):
            for p in (0, 1):
                pltpu.make_async_copy(idx_hbm.at[pl.ds(0, CHUNK)],
                                      idx_v[p], isem[p]).wait()

                @pl.when(jj > 0)
                def _():
                    base = ((2 * jj + p - 2) * NW + wid) * CHUNK
                    pltpu.make_async_copy(row_v[p], out_hbm.at[pl.ds(base, CHUNK)],
                                          wsem[p]).wait()
                pltpu.async_copy(node_hbm.at[idx_v[p]], row_v[p], gsem[p])
            for p in (0, 1):
                pltpu.make_async_copy(node_hbm.at[idx_v[p]], row_v[p],
                                      gsem[p]).wait()
                base = ((2 * jj + p) * NW + wid) * CHUNK
                pltpu.async_copy(row_v[p], out_hbm.at[pl.ds(base, CHUNK)], wsem[p])
                load_idx(p, jnp.minimum(2 * jj + p + 2, BASE_CH - 1))
            return 0

        lax.fori_loop(0, npair, body, 0)
        for p in (0, 1):
            base = ((BASE_CH - 2 + p) * NW + wid) * CHUNK
            pltpu.make_async_copy(row_v[p], out_hbm.at[pl.ds(base, CHUNK)],
                                  wsem[p]).wait()
            pltpu.make_async_copy(idx_hbm.at[pl.ds(0, CHUNK)],
                                  idx_v[p], isem[p]).wait()

        @pl.when(wid < REM_CH)
        def _():
            base = (BASE_CH * NW + wid) * CHUNK
            pltpu.sync_copy(idx_hbm.at[pl.ds(base, CHUNK)], idx_v0)
            pltpu.async_copy(node_hbm.at[idx_v0], row_v0, gsem0).wait()
            pltpu.sync_copy(row_v0, out_hbm.at[pl.ds(base, CHUNK)])


@functools.partial(jax.jit, static_argnames=())
def _sc_gather(node_fea, idx1, idx2):
    mesh = plsc.VectorSubcoreMesh(core_axis_name="c", subcore_axis_name="s",
                                  num_cores=NC, num_subcores=NS)
    return pl.kernel(
        _sc_gather_body,
        out_type=(jax.ShapeDtypeStruct((E, DV), jnp.float32),
                  jax.ShapeDtypeStruct((E, DV), jnp.float32)),
        mesh=mesh,
        scratch_types=[
            pltpu.VMEM((CHUNK,), jnp.int32),
            pltpu.VMEM((CHUNK,), jnp.int32),
            pltpu.VMEM((CHUNK, DV), jnp.float32),
            pltpu.VMEM((CHUNK, DV), jnp.float32),
            pltpu.SemaphoreType.DMA,
            pltpu.SemaphoreType.DMA,
            pltpu.SemaphoreType.DMA,
            pltpu.SemaphoreType.DMA,
            pltpu.SemaphoreType.DMA,
            pltpu.SemaphoreType.DMA,
        ],
    )(node_fea, idx1, idx2)


def _tc1_body(g1, g2, ef, w1a, w1b, w1c, b1, w2, b2, w3, b3,
              h_out, st_out):
    i = pl.program_id(0)
    x = jnp.dot(g1[...], w1a[...], preferred_element_type=jnp.float32)
    x += jnp.dot(g2[...], w1b[...], preferred_element_type=jnp.float32)
    x += jnp.dot(ef[...], w1c[...], preferred_element_type=jnp.float32)
    x = _lrelu(x + b1[...])
    x = _lrelu(jnp.dot(x, w2[...], preferred_element_type=jnp.float32) + b2[...])
    h = jnp.dot(x, w3[...], preferred_element_type=jnp.float32) + b3[...]
    h_out[...] = h

    @pl.when(i == 0)
    def _():
        st_out[...] = jnp.zeros_like(st_out)
    st_out[0:1, :] += jnp.sum(h, axis=0, keepdims=True)
    st_out[1:2, :] += jnp.sum(h * h, axis=0, keepdims=True)


def _tc1(g1, g2, edge_fea, pw):
    wspec = lambda shape: pl.BlockSpec(shape, lambda i: (0, 0))
    return pl.pallas_call(
        _tc1_body,
        grid=(GE,),
        in_specs=[
            pl.BlockSpec((KE, DV), lambda i: (i, 0)),
            pl.BlockSpec((KE, DV), lambda i: (i, 0)),
            pl.BlockSpec((KE, DE), lambda i: (i, 0)),
            wspec((DV, DE)), wspec((DV, DE)), wspec((DE, DE)), wspec((1, DE)),
            wspec((DE, DE)), wspec((1, DE)),
            wspec((DE, DE)), wspec((1, DE)),
        ],
        out_specs=[
            pl.BlockSpec((KE, DE), lambda i: (i, 0)),
            pl.BlockSpec((2, DE), lambda i: (0, 0)),
        ],
        out_shape=[
            jax.ShapeDtypeStruct((E, DE), jnp.float32),
            jax.ShapeDtypeStruct((2, DE), jnp.float32),
        ],
        compiler_params=pltpu.CompilerParams(
            dimension_semantics=("arbitrary",)),
    )(g1, g2, edge_fea, *pw)


def _tc2_body(g1, g2, h, ef, st, bnw, bnb,
              a1a, a1b, a1e, ab1, a2, ab2,
              m1a, m1b, m1e, mb1, m2, mb2,
              x_out, enew_out):
    mean = st[0:1, :] * (1.0 / E)
    var = st[1:2, :] * (1.0 / E) - mean * mean
    inv = lax.rsqrt(var + 1e-5)
    scale = bnw[...] * inv
    shift = bnb[...] - mean * scale

    ek = h[...] * scale + shift
    enew_out[...] = ef[...] + ek

    pre = jnp.dot(g1[...], a1a[...], preferred_element_type=jnp.float32)
    pre += jnp.dot(g2[...], a1b[...], preferred_element_type=jnp.float32)
    pre += jnp.dot(ek, a1e[...], preferred_element_type=jnp.float32)
    a = _lrelu(pre + ab1[...])
    sij = jnp.dot(a, a2[...], preferred_element_type=jnp.float32) + ab2[...]
    ex = jnp.exp(sij)

    pre = jnp.dot(g1[...], m1a[...], preferred_element_type=jnp.float32)
    pre += jnp.dot(g2[...], m1b[...], preferred_element_type=jnp.float32)
    pre += jnp.dot(ek, m1e[...], preferred_element_type=jnp.float32)
    m = _lrelu(pre + mb1[...])
    mij = jnp.dot(m, m2[...], preferred_element_type=jnp.float32) + mb2[...]
    exm = ex * mij

    for g in range(H):
        x_out[g, :, :] = ex[:, g * DV:(g + 1) * DV]
        x_out[H + g, :, :] = exm[:, g * DV:(g + 1) * DV]


def _tc2(g1, g2, h, edge_fea, stats, pw, off):
    wspec = lambda shape: pl.BlockSpec(shape, lambda i: (0, 0))
    return pl.pallas_call(
        _tc2_body,
        grid=(GE // 2,),
        in_specs=[
            pl.BlockSpec((KE, DV), lambda i: (i + off, 0)),
            pl.BlockSpec((KE, DV), lambda i: (i + off, 0)),
            pl.BlockSpec((KE, DE), lambda i: (i + off, 0)),
            pl.BlockSpec((KE, DE), lambda i: (i + off, 0)),
            wspec((2, DE)), wspec((1, DE)), wspec((1, DE)),
            wspec((DV, DV)), wspec((DV, DV)), wspec((DE, DV)), wspec((1, DV)),
            wspec((DV, FF)), wspec((1, FF)),
            wspec((DV, DV)), wspec((DV, DV)), wspec((DE, DV)), wspec((1, DV)),
            wspec((DV, FF)), wspec((1, FF)),
        ],
        out_specs=[
            pl.BlockSpec((2 * H, KE, DV), lambda i: (0, i, 0)),
            pl.BlockSpec((KE, DE), lambda i: (i, 0)),
        ],
        out_shape=[
            jax.ShapeDtypeStruct((2 * H, EH, DV), jnp.float32),
            jax.ShapeDtypeStruct((EH, DE), jnp.float32),
        ],
        compiler_params=pltpu.CompilerParams(
            dimension_semantics=("arbitrary",)),
    )(g1, g2, h, edge_fea, stats, *pw)


def _sc_scatter_body(x_hbm, idx_hbm, zeros_hbm, dn_hbm,
                     idx_v0, idx_v1, data_v0, data_v1, acc,
                     isem0, isem1, dsem0, dsem1, ssem0, ssem1):
    cid = lax.axis_index("c")
    sid = lax.axis_index("s")
    idx_v = (idx_v0, idx_v1)
    data_v = (data_v0, data_v1)
    isem = (isem0, isem1)
    dsem = (dsem0, dsem1)
    ssem = (ssem0, ssem1)
    npair = SC_BASE // 2

    for k in range(H):
        g = cid * H + k
        pltpu.sync_copy(zeros_hbm, acc.at[pl.ds(sid * STRIPE, STRIPE)])

        @pl.when(sid == NS - 1)
        def _():
            pltpu.sync_copy(zeros_hbm.at[pl.ds(0, STRIPE_TAIL)],
                            acc.at[pl.ds(STRIPE * NS, STRIPE_TAIL)])
        plsc.subcore_barrier()

        def load(p, j):
            base = (j * NS + sid) * CHUNK
            pltpu.async_copy(idx_hbm.at[pl.ds(base, CHUNK)], idx_v[p], isem[p])
            pltpu.async_copy(x_hbm.at[g, pl.ds(base, CHUNK)], data_v[p], dsem[p])

        def wait_load(p):
            pltpu.make_async_copy(idx_hbm.at[pl.ds(0, CHUNK)],
                                  idx_v[p], isem[p]).wait()
            pltpu.make_async_copy(x_hbm.at[g, pl.ds(0, CHUNK)],
                                  data_v[p], dsem[p]).wait()

        load(0, 0)
        load(1, 1)

        def body(jj, _):
            wait_load(0)
            pltpu.async_copy(data_v0, acc.at[idx_v0], ssem0, add=True)
            wait_load(1)
            pltpu.async_copy(data_v1, acc.at[idx_v1], ssem1, add=True)
            pltpu.make_async_copy(data_v0, acc.at[idx_v0], ssem0).wait()
            load(0, jnp.minimum(2 * jj + 2, SC_BASE - 1))
            pltpu.make_async_copy(data_v1, acc.at[idx_v1], ssem1).wait()
            load(1, jnp.minimum(2 * jj + 3, SC_BASE - 1))
            return 0

        lax.fori_loop(0, npair, body, 0)
        wait_load(0)
        wait_load(1)

        @pl.when(sid < SC_REM)
        def _():
            base = (SC_BASE * NS + sid) * CHUNK
            pltpu.sync_copy(idx_hbm.at[pl.ds(base, CHUNK)], idx_v0)
            pltpu.sync_copy(x_hbm.at[g, pl.ds(base, CHUNK)], data_v0)
            pltpu.sync_copy(data_v0, acc.at[idx_v0], add=True)
        plsc.subcore_barrier()

        pltpu.sync_copy(acc.at[pl.ds(sid * STRIPE, STRIPE)],
                        dn_hbm.at[g, pl.ds(sid * STRIPE, STRIPE)])

        @pl.when(sid == NS - 1)
        def _():
            pltpu.sync_copy(acc.at[pl.ds(STRIPE * NS, STRIPE_TAIL)],
                            dn_hbm.at[g, pl.ds(STRIPE * NS, STRIPE_TAIL)])
        plsc.subcore_barrier()


def _sc_scatter(x, idx1, zeros):
    mesh = plsc.VectorSubcoreMesh(core_axis_name="c", subcore_axis_name="s",
                                  num_cores=NC, num_subcores=NS)
    return pl.kernel(
        _sc_scatter_body,
        name="sc_scatter",
        out_type=jax.ShapeDtypeStruct((2 * H, B, DV), jnp.float32),
        mesh=mesh,
        scratch_types=[
            pltpu.VMEM((CHUNK,), jnp.int32),
            pltpu.VMEM((CHUNK,), jnp.int32),
            pltpu.VMEM((CHUNK, DV), jnp.float32),
            pltpu.VMEM((CHUNK, DV), jnp.float32),
            pltpu.VMEM_SHARED((B, DV), jnp.float32),
            pltpu.SemaphoreType.DMA,
            pltpu.SemaphoreType.DMA,
            pltpu.SemaphoreType.DMA,
            pltpu.SemaphoreType.DMA,
            pltpu.SemaphoreType.DMA,
            pltpu.SemaphoreType.DMA,
        ],
    )(x, idx1, zeros)


def _tc3a_body(dna, dnb, wo, wob, y_out, st_out):
    i = pl.program_id(0)
    cols = []
    for g in range(H):
        den = dna[g, :, :] + dnb[g, :, :]
        num = dna[H + g, :, :] + dnb[H + g, :, :]
        cols.append(jnp.where(den > 0, num / den, 0.0))
    msg = jnp.concatenate(cols, axis=1)
    y = jnp.dot(msg, wo[...], preferred_element_type=jnp.float32) + wob[...]
    y_out[...] = y

    @pl.when(i == 0)
    def _():
        st_out[...] = jnp.zeros_like(st_out)
    st_out[0:1, :] += jnp.sum(y, axis=0, keepdims=True)
    st_out[1:2, :] += jnp.sum(y * y, axis=0, keepdims=True)


def _tc3a(dna, dnb, woT, wob):
    wspec = lambda shape: pl.BlockSpec(shape, lambda i: (0, 0))
    return pl.pallas_call(
        _tc3a_body,
        grid=(GN,),
        in_specs=[
            pl.BlockSpec((2 * H, KN, DV), lambda i: (0, i, 0)),
            pl.BlockSpec((2 * H, KN, DV), lambda i: (0, i, 0)),
            wspec((FF, DV)), wspec((1, DV)),
        ],
        out_specs=[
            pl.BlockSpec((KN, DV), lambda i: (i, 0)),
            pl.BlockSpec((2, DV), lambda i: (0, 0)),
        ],
        out_shape=[
            jax.ShapeDtypeStruct((B, DV), jnp.float32),
            jax.ShapeDtypeStruct((2, DV), jnp.float32),
        ],
        compiler_params=pltpu.CompilerParams(
            dimension_semantics=("arbitrary",)),
    )(dna, dnb, woT, wob)


def _tc3b_body(y, node, st, bnw, bnb, out):
    mean = st[0:1, :] * (1.0 / B)
    var = st[1:2, :] * (1.0 / B) - mean * mean
    inv = lax.rsqrt(var + 1e-5)
    scale = bnw[...] * inv
    shift = bnb[...] - mean * scale
    out[...] = node[...] + y[...] * scale + shift


def _tc3b(y, node_fea, stats, bnw, bnb):
    wspec = lambda shape: pl.BlockSpec(shape, lambda i: (0, 0))
    return pl.pallas_call(
        _tc3b_body,
        grid=(GN,),
        in_specs=[
            pl.BlockSpec((KN, DV), lambda i: (i, 0)),
            pl.BlockSpec((KN, DV), lambda i: (i, 0)),
            wspec((2, DV)), wspec((1, DV)), wspec((1, DV)),
        ],
        out_specs=pl.BlockSpec((KN, DV), lambda i: (i, 0)),
        out_shape=jax.ShapeDtypeStruct((B, DV), jnp.float32),
        compiler_params=pltpu.CompilerParams(
            dimension_semantics=("arbitrary",)),
    )(y, node_fea, stats, bnw, bnb)


def kernel(node_fea, idx1, idx2, edge_fea,
           pe_w1, pe_b1, pe_w2, pe_b2, pe_w3, pe_b3,
           fa_w1, fa_b1, fa_w2, fa_b2,
           fm_w1, fm_b1, fm_w2, fm_b2,
           wo_w, wo_b, bn1_w, bn1_b, bn2_w, bn2_b):
    f32 = jnp.float32
    idx1 = idx1.astype(jnp.int32)
    idx2 = idx2.astype(jnp.int32)

    g1, g2 = _sc_gather(node_fea, idx1, idx2)

    pe = (pe_w1[:, :DV].T, pe_w1[:, DV:2 * DV].T, pe_w1[:, 2 * DV:].T,
          pe_b1.reshape(1, DE),
          pe_w2.T, pe_b2.reshape(1, DE),
          pe_w3.T, pe_b3.reshape(1, DE))
    h, stats1 = _tc1(g1, g2, edge_fea, pe)

    mw = (fa_w1[:, :DV].T, fa_w1[:, DV:2 * DV].T, fa_w1[:, 2 * DV:].T,
          fa_b1.reshape(1, DV), fa_w2.T, fa_b2.reshape(1, FF),
          fm_w1[:, :DV].T, fm_w1[:, DV:2 * DV].T, fm_w1[:, 2 * DV:].T,
          fm_b1.reshape(1, DV), fm_w2.T, fm_b2.reshape(1, FF))
    pw2 = (bn1_w.reshape(1, DE), bn1_b.reshape(1, DE)) + mw
    x_a, enew_a = _tc2(g1, g2, h, edge_fea, stats1, pw2, 0)
    x_b, enew_b = _tc2(g1, g2, h, edge_fea, stats1, pw2, GE // 2)
    edge_new = jnp.concatenate([enew_a, enew_b], axis=0)

    zeros = jnp.zeros((STRIPE, DV), f32)
    dn_a = _sc_scatter(x_a, idx1[:EH], zeros)
    dn_b = _sc_scatter(x_b, idx1[EH:], zeros)

    y, stats2 = _tc3a(dn_a, dn_b, wo_w.T, wo_b.reshape(1, DV))
    node_new = _tc3b(y, node_fea, stats2,
                     bn2_w.reshape(1, DV), bn2_b.reshape(1, DV))
    return (node_new, edge_new)


# speedup vs baseline: 1.0711x; 1.0210x over previous
import functools

import jax
import jax.numpy as jnp
from jax import lax
from jax.experimental import pallas as pl
from jax.experimental.pallas import tpu as pltpu
from jax.experimental.pallas import tpu_sc as plsc

B = 10000
E = 320000
DV = 128
DE = 16
H = 4
DT = 2 * DV + DE
FF = H * DV

NC = 2
NS = 16
CHUNK = 128
NCHUNKS = E // CHUNK
BASE_CH = 39
REM_CH = 2
EH = E // 2
NCHUNKS_H = EH // CHUNK
SC_BASE = NCHUNKS_H // NS
SC_REM = NCHUNKS_H - SC_BASE * NS
STRIPE = 624
STRIPE_TAIL = B - STRIPE * NS

KE = 2000
GE = E // KE
KN = 1000
GN = B // KN


def _lrelu(x):
    return jnp.where(x >= 0, x, 0.2 * x)


def _sc_gather_body(node_hbm, idx1_hbm, idx2_hbm, g1_hbm, g2_hbm,
                    idx_v0, idx_v1, row_v0, row_v1,
                    isem0, isem1, gsem0, gsem1, wsem0, wsem1):
    cid = lax.axis_index("c")
    sid = lax.axis_index("s")
    wid = sid * NC + cid
    NW = NC * NS
    idx_v = (idx_v0, idx_v1)
    row_v = (row_v0, row_v1)
    isem = (isem0, isem1)
    gsem = (gsem0, gsem1)
    wsem = (wsem0, wsem1)
    npair = BASE_CH // 2

    for idx_hbm, out_hbm in ((idx1_hbm, g1_hbm), (idx2_hbm, g2_hbm)):
        def load_idx(p, j):
            base = (j * NW + wid) * CHUNK
            pltpu.async_copy(idx_hbm.at[pl.ds(base, CHUNK)], idx_v[p], isem[p])

        load_idx(0, 0)
        load_idx(1, 1)

        def body(jj, _):
            for p in (0, 1):
                pltpu.make_async_copy(idx_hbm.at[pl.ds(0, CHUNK)],
                                      idx_v[p], isem[p]).wait()

                @pl.when(jj > 0)
                def _():
                    base = ((2 * jj + p - 2) * NW + wid) * CHUNK
                    pltpu.make_async_copy(row_v[p], out_hbm.at[pl.ds(base, CHUNK)],
                                          wsem[p]).wait()
                pltpu.async_copy(node_hbm.at[idx_v[p]], row_v[p], gsem[p])
            for p in (0, 1):
                pltpu.make_async_copy(node_hbm.at[idx_v[p]], row_v[p],
                                      gsem[p]).wait()
                base = ((2 * jj + p) * NW + wid) * CHUNK
                pltpu.async_copy(row_v[p], out_hbm.at[pl.ds(base, CHUNK)], wsem[p])
                load_idx(p, jnp.minimum(2 * jj + p + 2, BASE_CH - 1))
            return 0

        lax.fori_loop(0, npair, body, 0)
        for p in (0, 1):
            base = ((2 * npair - 2 + p) * NW + wid) * CHUNK
            pltpu.make_async_copy(row_v[p], out_hbm.at[pl.ds(base, CHUNK)],
                                  wsem[p]).wait()
            pltpu.make_async_copy(idx_hbm.at[pl.ds(0, CHUNK)],
                                  idx_v[p], isem[p]).wait()

        def tail(j):
            base = (j * NW + wid) * CHUNK
            pltpu.sync_copy(idx_hbm.at[pl.ds(base, CHUNK)], idx_v0)
            pltpu.async_copy(node_hbm.at[idx_v0], row_v0, gsem0).wait()
            pltpu.sync_copy(row_v0, out_hbm.at[pl.ds(base, CHUNK)])

        tail(2 * npair)

        @pl.when(wid < REM_CH)
        def _():
            tail(BASE_CH)


@functools.partial(jax.jit, static_argnames=())
def _sc_gather(node_fea, idx1, idx2):
    mesh = plsc.VectorSubcoreMesh(core_axis_name="c", subcore_axis_name="s",
                                  num_cores=NC, num_subcores=NS)
    return pl.kernel(
        _sc_gather_body,
        name="sc_gather",
        out_type=(jax.ShapeDtypeStruct((EH, DV), jnp.float32),
                  jax.ShapeDtypeStruct((EH, DV), jnp.float32)),
        mesh=mesh,
        scratch_types=[
            pltpu.VMEM((CHUNK,), jnp.int32),
            pltpu.VMEM((CHUNK,), jnp.int32),
            pltpu.VMEM((CHUNK, DV), jnp.float32),
            pltpu.VMEM((CHUNK, DV), jnp.float32),
            pltpu.SemaphoreType.DMA,
            pltpu.SemaphoreType.DMA,
            pltpu.SemaphoreType.DMA,
            pltpu.SemaphoreType.DMA,
            pltpu.SemaphoreType.DMA,
            pltpu.SemaphoreType.DMA,
        ],
    )(node_fea, idx1, idx2)


def _tc1_body(g1, g2, ef, w1a, w1b, w1c, b1, w2, b2, w3, b3,
              h_out, st_out):
    i = pl.program_id(0)
    x = jnp.dot(g1[...], w1a[...], preferred_element_type=jnp.float32)
    x += jnp.dot(g2[...], w1b[...], preferred_element_type=jnp.float32)
    x += jnp.dot(ef[...], w1c[...], preferred_element_type=jnp.float32)
    x = _lrelu(x + b1[...])
    x = _lrelu(jnp.dot(x, w2[...], preferred_element_type=jnp.float32) + b2[...])
    h = jnp.dot(x, w3[...], preferred_element_type=jnp.float32) + b3[...]
    h_out[...] = h

    @pl.when(i == 0)
    def _():
        st_out[...] = jnp.zeros_like(st_out)
    st_out[0:1, :] += jnp.sum(h, axis=0, keepdims=True)
    st_out[1:2, :] += jnp.sum(h * h, axis=0, keepdims=True)


def _tc1(g1, g2, edge_fea, pw, off):
    wspec = lambda shape: pl.BlockSpec(shape, lambda i: (0, 0))
    return pl.pallas_call(
        _tc1_body,
        grid=(GE // 2,),
        in_specs=[
            pl.BlockSpec((KE, DV), lambda i: (i, 0)),
            pl.BlockSpec((KE, DV), lambda i: (i, 0)),
            pl.BlockSpec((KE, DE), lambda i: (i + off, 0)),
            wspec((DV, DE)), wspec((DV, DE)), wspec((DE, DE)), wspec((1, DE)),
            wspec((DE, DE)), wspec((1, DE)),
            wspec((DE, DE)), wspec((1, DE)),
        ],
        out_specs=[
            pl.BlockSpec((KE, DE), lambda i: (i, 0)),
            pl.BlockSpec((2, DE), lambda i: (0, 0)),
        ],
        out_shape=[
            jax.ShapeDtypeStruct((EH, DE), jnp.float32),
            jax.ShapeDtypeStruct((2, DE), jnp.float32),
        ],
        compiler_params=pltpu.CompilerParams(
            dimension_semantics=("arbitrary",)),
    )(g1, g2, edge_fea, *pw)


def _tc2_body(g1, g2, h, ef, st_a, st_b, bnw, bnb,
              a1a, a1b, a1e, ab1, a2, ab2,
              m1a, m1b, m1e, mb1, m2, mb2,
              x_out, enew_out):
    st = st_a[...] + st_b[...]
    mean = st[0:1, :] * (1.0 / E)
    var = st[1:2, :] * (1.0 / E) - mean * mean
    inv = lax.rsqrt(var + 1e-5)
    scale = bnw[...] * inv
    shift = bnb[...] - mean * scale

    ek = h[...] * scale + shift
    enew_out[...] = ef[...] + ek

    pre = jnp.dot(g1[...], a1a[...], preferred_element_type=jnp.float32)
    pre += jnp.dot(g2[...], a1b[...], preferred_element_type=jnp.float32)
    pre += jnp.dot(ek, a1e[...], preferred_element_type=jnp.float32)
    a = _lrelu(pre + ab1[...])
    sij = jnp.dot(a, a2[...], preferred_element_type=jnp.float32) + ab2[...]
    ex = jnp.exp(sij)

    pre = jnp.dot(g1[...], m1a[...], preferred_element_type=jnp.float32)
    pre += jnp.dot(g2[...], m1b[...], preferred_element_type=jnp.float32)
    pre += jnp.dot(ek, m1e[...], preferred_element_type=jnp.float32)
    m = _lrelu(pre + mb1[...])
    mij = jnp.dot(m, m2[...], preferred_element_type=jnp.float32) + mb2[...]
    exm = ex * mij

    for g in range(H):
        x_out[g, :, :] = ex[:, g * DV:(g + 1) * DV]
        x_out[H + g, :, :] = exm[:, g * DV:(g + 1) * DV]


def _tc2(g1, g2, h, edge_fea, st_a, st_b, pw, off):
    wspec = lambda shape: pl.BlockSpec(shape, lambda i: (0, 0))
    return pl.pallas_call(
        _tc2_body,
        grid=(GE // 2,),
        in_specs=[
            pl.BlockSpec((KE, DV), lambda i: (i, 0)),
            pl.BlockSpec((KE, DV), lambda i: (i, 0)),
            pl.BlockSpec((KE, DE), lambda i: (i, 0)),
            pl.BlockSpec((KE, DE), lambda i: (i + off, 0)),
            wspec((2, DE)), wspec((2, DE)), wspec((1, DE)), wspec((1, DE)),
            wspec((DV, DV)), wspec((DV, DV)), wspec((DE, DV)), wspec((1, DV)),
            wspec((DV, FF)), wspec((1, FF)),
            wspec((DV, DV)), wspec((DV, DV)), wspec((DE, DV)), wspec((1, DV)),
            wspec((DV, FF)), wspec((1, FF)),
        ],
        out_specs=[
            pl.BlockSpec((2 * H, KE, DV), lambda i: (0, i, 0)),
            pl.BlockSpec((KE, DE), lambda i: (i, 0)),
        ],
        out_shape=[
            jax.ShapeDtypeStruct((2 * H, EH, DV), jnp.float32),
            jax.ShapeDtypeStruct((EH, DE), jnp.float32),
        ],
        compiler_params=pltpu.CompilerParams(
            dimension_semantics=("arbitrary",)),
    )(g1, g2, h, edge_fea, st_a, st_b, *pw)


def _sc_scatter_body(x_hbm, idx_hbm, zeros_hbm, dn_hbm,
                     idx_v0, idx_v1, data_v0, data_v1, acc,
                     isem0, isem1, dsem0, dsem1, ssem0, ssem1):
    cid = lax.axis_index("c")
    sid = lax.axis_index("s")
    idx_v = (idx_v0, idx_v1)
    data_v = (data_v0, data_v1)
    isem = (isem0, isem1)
    dsem = (dsem0, dsem1)
    ssem = (ssem0, ssem1)
    npair = SC_BASE // 2

    for k in range(H):
        g = cid * H + k
        pltpu.sync_copy(zeros_hbm, acc.at[pl.ds(sid * STRIPE, STRIPE)])

        @pl.when(sid == NS - 1)
        def _():
            pltpu.sync_copy(zeros_hbm.at[pl.ds(0, STRIPE_TAIL)],
                            acc.at[pl.ds(STRIPE * NS, STRIPE_TAIL)])
        plsc.subcore_barrier()

        def load(p, j):
            base = (j * NS + sid) * CHUNK
            pltpu.async_copy(idx_hbm.at[pl.ds(base, CHUNK)], idx_v[p], isem[p])
            pltpu.async_copy(x_hbm.at[g, pl.ds(base, CHUNK)], data_v[p], dsem[p])

        def wait_load(p):
            pltpu.make_async_copy(idx_hbm.at[pl.ds(0, CHUNK)],
                                  idx_v[p], isem[p]).wait()
            pltpu.make_async_copy(x_hbm.at[g, pl.ds(0, CHUNK)],
                                  data_v[p], dsem[p]).wait()

        load(0, 0)
        load(1, 1)

        def body(jj, _):
            wait_load(0)
            pltpu.async_copy(data_v0, acc.at[idx_v0], ssem0, add=True)
            wait_load(1)
            pltpu.async_copy(data_v1, acc.at[idx_v1], ssem1, add=True)
            pltpu.make_async_copy(data_v0, acc.at[idx_v0], ssem0).wait()
            load(0, jnp.minimum(2 * jj + 2, SC_BASE - 1))
            pltpu.make_async_copy(data_v1, acc.at[idx_v1], ssem1).wait()
            load(1, jnp.minimum(2 * jj + 3, SC_BASE - 1))
            return 0

        lax.fori_loop(0, npair, body, 0)
        wait_load(0)
        wait_load(1)

        @pl.when(sid < SC_REM)
        def _():
            base = (SC_BASE * NS + sid) * CHUNK
            pltpu.sync_copy(idx_hbm.at[pl.ds(base, CHUNK)], idx_v0)
            pltpu.sync_copy(x_hbm.at[g, pl.ds(base, CHUNK)], data_v0)
            pltpu.sync_copy(data_v0, acc.at[idx_v0], add=True)
        plsc.subcore_barrier()

        pltpu.sync_copy(acc.at[pl.ds(sid * STRIPE, STRIPE)],
                        dn_hbm.at[g, pl.ds(sid * STRIPE, STRIPE)])

        @pl.when(sid == NS - 1)
        def _():
            pltpu.sync_copy(acc.at[pl.ds(STRIPE * NS, STRIPE_TAIL)],
                            dn_hbm.at[g, pl.ds(STRIPE * NS, STRIPE_TAIL)])
        plsc.subcore_barrier()


def _sc_scatter(x, idx1, zeros):
    mesh = plsc.VectorSubcoreMesh(core_axis_name="c", subcore_axis_name="s",
                                  num_cores=NC, num_subcores=NS)
    return pl.kernel(
        _sc_scatter_body,
        name="sc_scatter",
        out_type=jax.ShapeDtypeStruct((2 * H, B, DV), jnp.float32),
        mesh=mesh,
        scratch_types=[
            pltpu.VMEM((CHUNK,), jnp.int32),
            pltpu.VMEM((CHUNK,), jnp.int32),
            pltpu.VMEM((CHUNK, DV), jnp.float32),
            pltpu.VMEM((CHUNK, DV), jnp.float32),
            pltpu.VMEM_SHARED((B, DV), jnp.float32),
            pltpu.SemaphoreType.DMA,
            pltpu.SemaphoreType.DMA,
            pltpu.SemaphoreType.DMA,
            pltpu.SemaphoreType.DMA,
            pltpu.SemaphoreType.DMA,
            pltpu.SemaphoreType.DMA,
        ],
    )(x, idx1, zeros)


def _tc3a_body(dna, dnb, wo, wob, y_out, st_out):
    i = pl.program_id(0)
    cols = []
    for g in range(H):
        den = dna[g, :, :] + dnb[g, :, :]
        num = dna[H + g, :, :] + dnb[H + g, :, :]
        cols.append(jnp.where(den > 0, num / den, 0.0))
    msg = jnp.concatenate(cols, axis=1)
    y = jnp.dot(msg, wo[...], preferred_element_type=jnp.float32) + wob[...]
    y_out[...] = y

    @pl.when(i == 0)
    def _():
        st_out[...] = jnp.zeros_like(st_out)
    st_out[0:1, :] += jnp.sum(y, axis=0, keepdims=True)
    st_out[1:2, :] += jnp.sum(y * y, axis=0, keepdims=True)


def _tc3a(dna, dnb, woT, wob):
    wspec = lambda shape: pl.BlockSpec(shape, lambda i: (0, 0))
    return pl.pallas_call(
        _tc3a_body,
        grid=(GN,),
        in_specs=[
            pl.BlockSpec((2 * H, KN, DV), lambda i: (0, i, 0)),
            pl.BlockSpec((2 * H, KN, DV), lambda i: (0, i, 0)),
            wspec((FF, DV)), wspec((1, DV)),
        ],
        out_specs=[
            pl.BlockSpec((KN, DV), lambda i: (i, 0)),
            pl.BlockSpec((2, DV), lambda i: (0, 0)),
        ],
        out_shape=[
            jax.ShapeDtypeStruct((B, DV), jnp.float32),
            jax.ShapeDtypeStruct((2, DV), jnp.float32),
        ],
        compiler_params=pltpu.CompilerParams(
            dimension_semantics=("arbitrary",)),
    )(dna, dnb, woT, wob)


def _tc3b_body(y, node, st, bnw, bnb, out):
    mean = st[0:1, :] * (1.0 / B)
    var = st[1:2, :] * (1.0 / B) - mean * mean
    inv = lax.rsqrt(var + 1e-5)
    scale = bnw[...] * inv
    shift = bnb[...] - mean * scale
    out[...] = node[...] + y[...] * scale + shift


def _tc3b(y, node_fea, stats, bnw, bnb):
    wspec = lambda shape: pl.BlockSpec(shape, lambda i: (0, 0))
    return pl.pallas_call(
        _tc3b_body,
        grid=(GN,),
        in_specs=[
            pl.BlockSpec((KN, DV), lambda i: (i, 0)),
            pl.BlockSpec((KN, DV), lambda i: (i, 0)),
            wspec((2, DV)), wspec((1, DV)), wspec((1, DV)),
        ],
        out_specs=pl.BlockSpec((KN, DV), lambda i: (i, 0)),
        out_shape=jax.ShapeDtypeStruct((B, DV), jnp.float32),
        compiler_params=pltpu.CompilerParams(
            dimension_semantics=("arbitrary",)),
    )(y, node_fea, stats, bnw, bnb)


def kernel(node_fea, idx1, idx2, edge_fea,
           pe_w1, pe_b1, pe_w2, pe_b2, pe_w3, pe_b3,
           fa_w1, fa_b1, fa_w2, fa_b2,
           fm_w1, fm_b1, fm_w2, fm_b2,
           wo_w, wo_b, bn1_w, bn1_b, bn2_w, bn2_b):
    f32 = jnp.float32
    idx1 = idx1.astype(jnp.int32)
    idx2 = idx2.astype(jnp.int32)

    g1a, g2a = _sc_gather(node_fea, idx1[:EH], idx2[:EH])
    g1b, g2b = _sc_gather(node_fea, idx1[EH:], idx2[EH:])

    pe = (pe_w1[:, :DV].T, pe_w1[:, DV:2 * DV].T, pe_w1[:, 2 * DV:].T,
          pe_b1.reshape(1, DE),
          pe_w2.T, pe_b2.reshape(1, DE),
          pe_w3.T, pe_b3.reshape(1, DE))
    h_a, st_a = _tc1(g1a, g2a, edge_fea, pe, 0)
    h_b, st_b = _tc1(g1b, g2b, edge_fea, pe, GE // 2)

    mw = (fa_w1[:, :DV].T, fa_w1[:, DV:2 * DV].T, fa_w1[:, 2 * DV:].T,
          fa_b1.reshape(1, DV), fa_w2.T, fa_b2.reshape(1, FF),
          fm_w1[:, :DV].T, fm_w1[:, DV:2 * DV].T, fm_w1[:, 2 * DV:].T,
          fm_b1.reshape(1, DV), fm_w2.T, fm_b2.reshape(1, FF))
    pw2 = (bn1_w.reshape(1, DE), bn1_b.reshape(1, DE)) + mw
    x_a, enew_a = _tc2(g1a, g2a, h_a, edge_fea, st_a, st_b, pw2, 0)
    x_b, enew_b = _tc2(g1b, g2b, h_b, edge_fea, st_a, st_b, pw2, GE // 2)
    edge_new = jnp.concatenate([enew_a, enew_b], axis=0)

    zeros = jnp.zeros((STRIPE, DV), f32)
    dn_a = _sc_scatter(x_a, idx1[:EH], zeros)
    dn_b = _sc_scatter(x_b, idx1[EH:], zeros)

    y, stats2 = _tc3a(dn_a, dn_b, wo_w.T, wo_b.reshape(1, DV))
    node_new = _tc3b(y, node_fea, stats2,
                     bn2_w.reshape(1, DV), bn2_b.reshape(1, DV))
    return (node_new, edge_new)
